# baseline (speedup 1.0000x reference)
import functools

import jax
import jax.numpy as jnp
from jax import lax
from jax.experimental import pallas as pl
from jax.experimental.pallas import tpu as pltpu
from jax.experimental.pallas import tpu_sc as plsc

VOCAB = 1000000
D = 64
B_TOTAL = 4096 * 200
ROW = 128
N_ROWS = B_TOTAL // ROW
NC, NS = 2, 16
NW = NC * NS
ROWS_PER_W = N_ROWS // NW
G = 8
CHUNK = G * ROW
N_ITER = ROWS_PER_W // G

_mesh = plsc.VectorSubcoreMesh(core_axis_name="c", subcore_axis_name="s")


@functools.partial(
    pl.kernel,
    out_type=jax.ShapeDtypeStruct((B_TOTAL, D), jnp.float32),
    mesh=_mesh,
    compiler_params=pltpu.CompilerParams(use_tc_tiling_on_sc=False),
    scratch_types=[
        pltpu.VMEM((G, ROW), jnp.int32),
        pltpu.VMEM((CHUNK, D), jnp.float32),
        pltpu.SemaphoreType.DMA,
    ],
)
def _embed_gather(idx_hbm, table_hbm, out_hbm, idx_v, rows_v, sem):
    wid = lax.axis_index("s") * NC + lax.axis_index("c")
    row0 = wid * ROWS_PER_W

    def step(i, carry):
        r = row0 + i * G
        pltpu.sync_copy(idx_hbm.at[pl.ds(r, G)], idx_v)
        copies = [
            pltpu.async_copy(
                table_hbm.at[idx_v.at[j]],
                rows_v.at[pl.ds(j * ROW, ROW)],
                sem,
            )
            for j in range(G)
        ]
        for cp in copies:
            cp.wait()
        pltpu.sync_copy(rows_v, out_hbm.at[pl.ds(r * ROW, CHUNK)])
        return carry

    lax.fori_loop(0, N_ITER, step, 0)


def kernel(x, table):
    idx = x.reshape(N_ROWS, ROW)
    out = _embed_gather(idx, table)
    return out.reshape(4096, 200, D)


# speedup vs baseline: 1.0124x; 1.0124x over previous
import functools

import jax
import jax.numpy as jnp
from jax import lax
from jax.experimental import pallas as pl
from jax.experimental.pallas import tpu as pltpu
from jax.experimental.pallas import tpu_sc as plsc

VOCAB = 1000000
D = 64
B_TOTAL = 4096 * 200
ROW = 128
N_ROWS = B_TOTAL // ROW
NC, NS = 2, 16
NW = NC * NS
ROWS_PER_W = N_ROWS // NW
CH_ROWS = 4
CHUNK = CH_ROWS * ROW
N_CH = ROWS_PER_W // CH_ROWS

_mesh = plsc.VectorSubcoreMesh(core_axis_name="c", subcore_axis_name="s")


@functools.partial(
    pl.kernel,
    out_type=jax.ShapeDtypeStruct((B_TOTAL, D), jnp.float32),
    mesh=_mesh,
    compiler_params=pltpu.CompilerParams(use_tc_tiling_on_sc=False),
    scratch_types=[
        pltpu.VMEM((ROWS_PER_W, ROW), jnp.int32),
        pltpu.VMEM((CHUNK, D), jnp.float32),
        pltpu.VMEM((CHUNK, D), jnp.float32),
        pltpu.SemaphoreType.DMA,
        pltpu.SemaphoreType.DMA,
        pltpu.SemaphoreType.DMA,
        pltpu.SemaphoreType.DMA,
    ],
)
def _embed_gather(idx_hbm, table_hbm, out_hbm, idx_all, rows_a, rows_b,
                  semg_a, semg_b, sems_a, sems_b):
    wid = lax.axis_index("s") * NC + lax.axis_index("c")
    irow0 = wid * ROWS_PER_W
    orow0 = wid * ROWS_PER_W * ROW

    pltpu.sync_copy(idx_hbm.at[pl.ds(irow0, ROWS_PER_W)], idx_all)

    def fire_gather(c, buf, sem):
        for j in range(CH_ROWS):
            pltpu.async_copy(
                table_hbm.at[idx_all.at[c * CH_ROWS + j]],
                buf.at[pl.ds(j * ROW, ROW)],
                sem,
            )

    def wait_gather(c, buf, sem):
        for j in range(CH_ROWS):
            pltpu.make_async_copy(
                table_hbm.at[idx_all.at[c * CH_ROWS + j]],
                buf.at[pl.ds(j * ROW, ROW)],
                sem,
            ).wait()

    def fire_store(c, buf, sem):
        pltpu.async_copy(buf, out_hbm.at[pl.ds(orow0 + c * CHUNK, CHUNK)], sem)

    def wait_store(c, buf, sem):
        pltpu.make_async_copy(
            buf, out_hbm.at[pl.ds(orow0 + c * CHUNK, CHUNK)], sem
        ).wait()

    fire_gather(0, rows_a, semg_a)
    fire_gather(1, rows_b, semg_b)

    def step(k, carry):
        a = 2 * k
        b = 2 * k + 1
        wait_gather(a, rows_a, semg_a)
        fire_store(a, rows_a, sems_a)
        wait_gather(b, rows_b, semg_b)
        fire_store(b, rows_b, sems_b)

        @pl.when(k < N_CH // 2 - 1)
        def _refill():
            wait_store(a, rows_a, sems_a)
            fire_gather(a + 2, rows_a, semg_a)
            wait_store(b, rows_b, sems_b)
            fire_gather(b + 2, rows_b, semg_b)

        return carry

    lax.fori_loop(0, N_CH // 2, step, 0)

    wait_store(N_CH - 2, rows_a, sems_a)
    wait_store(N_CH - 1, rows_b, sems_b)


def kernel(x, table):
    idx = x.reshape(N_ROWS, ROW)
    out = _embed_gather(idx, table)
    return out.reshape(4096, 200, D)
